# baseline (speedup 1.0000x reference)
import jax
import jax.numpy as jnp
from jax import lax
from jax.experimental import pallas as pl
from jax.experimental.pallas import tpu as pltpu

N = 10000
K = 32
D = 128
C = 1000
CH = 400
NCH = N // CH
CHR = CH * K
R = 4


def _body(ws_ref, wn_ref, b_ref, fcw_ref, fcb_ref,
          x0_hbm, x1_hbm, out_hbm, sc_hbm,
          x1buf, x0buf, outbuf, scbuf,
          in_sem, in0_sem, out_sem, sc_sem):

    def in_copies(c, slot):
        return (
            pltpu.make_async_copy(
                x1_hbm.at[pl.ds(c * CHR, CHR)], x1buf.at[slot],
                in_sem.at[slot]),
            pltpu.make_async_copy(
                x0_hbm.at[pl.ds(c * CH, CH)], x0buf.at[slot],
                in0_sem.at[slot]),
        )

    def out_copies(c, slot):
        return (
            pltpu.make_async_copy(
                outbuf.at[slot], out_hbm.at[pl.ds(c * CH, CH)],
                out_sem.at[slot]),
            pltpu.make_async_copy(
                scbuf.at[slot], sc_hbm.at[pl.ds(c * CH, CH)],
                sc_sem.at[slot]),
        )

    for r in range(R):
        for cp in in_copies(r, r):
            cp.start()

    def step(c, carry):
        slot = lax.rem(c, R)
        for cp in in_copies(c, slot):
            cp.wait()

        @pl.when(c >= R)
        def _():
            for cp in out_copies(c - R, slot):
                cp.wait()

        x0b = x0buf[slot]
        mean = jnp.mean(x1buf[slot].reshape(CH, K, D), axis=1)
        out = (
            jnp.dot(x0b, ws_ref[...], preferred_element_type=jnp.float32)
            + jnp.dot(mean, wn_ref[...], preferred_element_type=jnp.float32)
            + b_ref[...]
            + x0b
        )
        outbuf[slot] = out
        scbuf[slot] = (
            jnp.dot(jnp.maximum(out, 0.0), fcw_ref[...],
                    preferred_element_type=jnp.float32)
            + fcb_ref[...]
        )
        for cp in out_copies(c, slot):
            cp.start()

        @pl.when(c + R < NCH)
        def _():
            for cp in in_copies(c + R, slot):
                cp.start()

        return carry

    lax.fori_loop(0, NCH, step, 0)

    for r in range(R):
        c = NCH - R + r
        for cp in out_copies(c, c % R):
            cp.wait()


def kernel(x0, x1, W_self, W_neigh, b, fc_W, fc_b):
    b2 = b.reshape(1, D)
    fcb2 = fc_b.reshape(1, C)
    out, scores = pl.pallas_call(
        _body,
        in_specs=[
            pl.BlockSpec((D, D), lambda: (0, 0)),
            pl.BlockSpec((D, D), lambda: (0, 0)),
            pl.BlockSpec((1, D), lambda: (0, 0)),
            pl.BlockSpec((D, C), lambda: (0, 0)),
            pl.BlockSpec((1, C), lambda: (0, 0)),
            pl.BlockSpec(memory_space=pl.ANY),
            pl.BlockSpec(memory_space=pl.ANY),
        ],
        out_specs=[
            pl.BlockSpec(memory_space=pl.ANY),
            pl.BlockSpec(memory_space=pl.ANY),
        ],
        out_shape=[
            jax.ShapeDtypeStruct((N, D), jnp.float32),
            jax.ShapeDtypeStruct((N, C), jnp.float32),
        ],
        scratch_shapes=[
            pltpu.VMEM((R, CHR, D), jnp.float32),
            pltpu.VMEM((R, CH, D), jnp.float32),
            pltpu.VMEM((R, CH, D), jnp.float32),
            pltpu.VMEM((R, CH, C), jnp.float32),
            pltpu.SemaphoreType.DMA((R,)),
            pltpu.SemaphoreType.DMA((R,)),
            pltpu.SemaphoreType.DMA((R,)),
            pltpu.SemaphoreType.DMA((R,)),
        ],
    )(W_self, W_neigh, b2, fc_W, fcb2, x0, x1)
    return (out, scores)
